# baseline (speedup 1.0000x reference)
import jax
import jax.numpy as jnp
from jax import lax
from jax.experimental import pallas as pl
from jax.experimental.pallas import tpu as pltpu
from jax.experimental.pallas import tpu_sc as plsc

B = 4096
L = 200
N = B * L
WDIM = 50
PDIM = 5
ODIM = 60
TPAD = 56
PLEN = 400

NC = 2
NS = 16
NW = NC * NS
PER_W = N // NW
C = 256
CW = C * ODIM
CHUNKS = PER_W // C
VGRP = C // 16


def _body(word_hbm, p1_hbm, p2_hbm, wt_hbm, p1t_hbm, p2t_hbm, out_hbm,
          widx, p1idx, p2idx, t56, t60, p1t, p2t, semi, semg, semo):
  wid = lax.axis_index("s") * NC + lax.axis_index("c")
  base = wid * PER_W
  pltpu.sync_copy(p1t_hbm, p1t)
  pltpu.sync_copy(p2t_hbm, p2t)

  def fire_idx(g, b):
    cb = pl.multiple_of(base + g * C, C)
    pltpu.async_copy(word_hbm.at[pl.ds(cb, C)], widx.at[b], semi[b])
    pltpu.async_copy(p1_hbm.at[pl.ds(cb, C)], p1idx.at[b], semi[b])
    pltpu.async_copy(p2_hbm.at[pl.ds(cb, C)], p2idx.at[b], semi[b])

  def wait_idx(b):
    pltpu.make_async_copy(word_hbm.at[pl.ds(0, C)], widx.at[b], semi[b]).wait()
    pltpu.make_async_copy(p1_hbm.at[pl.ds(0, C)], p1idx.at[b], semi[b]).wait()
    pltpu.make_async_copy(p2_hbm.at[pl.ds(0, C)], p2idx.at[b], semi[b]).wait()

  def fire_gathers(b):
    for j in range(C // 128):
      pltpu.async_copy(wt_hbm.at[widx.at[b, pl.ds(j * 128, 128)]],
                       t56.at[b, pl.ds(j * 128, 128)], semg[b])

  def wait_gathers(b):
    pltpu.make_async_copy(wt_hbm.at[pl.ds(0, C)], t56.at[b], semg[b]).wait()

  def fire_out(g, b):
    cw = pl.multiple_of((base + g * C) * ODIM, CW)
    pltpu.async_copy(t60.at[b], out_hbm.at[pl.ds(cw, CW)], semo[b])

  def wait_out(b):
    pltpu.make_async_copy(t60.at[b], out_hbm.at[pl.ds(0, CW)], semo[b]).wait()

  fire_idx(0, 0)
  fire_idx(1, 1)
  wait_idx(0)
  fire_gathers(0)

  iota = lax.iota(jnp.int32, 16)
  tailmask = (iota + 40 >= 48) & (iota + 40 < WDIM)

  @pl.loop(0, CHUNKS // 2)
  def _outer(go):
    for b in range(2):
      g = go * 2 + b
      nb = 1 - b

      @pl.when(g + 1 < CHUNKS)
      def _():
        wait_idx(nb)
        fire_gathers(nb)

      wait_gathers(b)

      @pl.when(g >= 2)
      def _():
        wait_out(b)

      @plsc.parallel_loop(0, C, unroll=4)
      def _tok(t):
        v0 = t56.at[b, t][pl.ds(0, 16)]
        v1 = t56.at[b, t][pl.ds(16, 16)]
        v2 = t56.at[b, t][pl.ds(32, 16)]
        v3 = t56.at[b, t][pl.ds(40, 16)]
        e0 = iota + t * ODIM
        plsc.store_scatter(t60.at[b], [e0], v0)
        plsc.store_scatter(t60.at[b], [e0 + 16], v1)
        plsc.store_scatter(t60.at[b], [e0 + 32], v2)
        plsc.store_scatter(t60.at[b], [e0 + 40], v3, mask=tailmask)

      @plsc.parallel_loop(0, VGRP, unroll=2)
      def _grp(i):
        ebase = (iota + i * 16) * ODIM + WDIM
        i1 = p1idx.at[b][pl.ds(i * 16, 16)] * PDIM
        i2 = p2idx.at[b][pl.ds(i * 16, 16)] * PDIM
        v1s = [plsc.load_gather(p1t, [i1 + jnp.full((16,), j, jnp.int32)])
               for j in range(PDIM)]
        v2s = [plsc.load_gather(p2t, [i2 + jnp.full((16,), j, jnp.int32)])
               for j in range(PDIM)]
        for j in range(PDIM):
          plsc.store_scatter(t60.at[b], [ebase + j], v1s[j])
          plsc.store_scatter(t60.at[b], [ebase + (PDIM + j)], v2s[j])

      fire_out(g, b)

      @pl.when(g + 2 < CHUNKS)
      def _():
        fire_idx(g + 2, b)

  wait_out(0)
  wait_out(1)


def kernel(word, pos1, pos2, word_table, pos1_table, pos2_table):
  mesh = plsc.VectorSubcoreMesh(core_axis_name="c", subcore_axis_name="s")
  run = pl.kernel(
      _body,
      out_type=jax.ShapeDtypeStruct((N * ODIM,), jnp.float32),
      mesh=mesh,
      scratch_types=[
          pltpu.VMEM((2, C), jnp.int32),
          pltpu.VMEM((2, C), jnp.int32),
          pltpu.VMEM((2, C), jnp.int32),
          pltpu.VMEM((2, C, TPAD), jnp.float32),
          pltpu.VMEM((2, CW), jnp.float32),
          pltpu.VMEM((PLEN * PDIM,), jnp.float32),
          pltpu.VMEM((PLEN * PDIM,), jnp.float32),
          [pltpu.SemaphoreType.DMA, pltpu.SemaphoreType.DMA],
          [pltpu.SemaphoreType.DMA, pltpu.SemaphoreType.DMA],
          [pltpu.SemaphoreType.DMA, pltpu.SemaphoreType.DMA],
      ],
      compiler_params=pltpu.CompilerParams(
          use_tc_tiling_on_sc=False, needs_layout_passes=False),
  )
  wt_pad = jnp.pad(word_table.astype(jnp.float32), ((0, 0), (0, TPAD - WDIM)))
  out = run(
      word.reshape(N).astype(jnp.int32),
      pos1.reshape(N).astype(jnp.int32),
      pos2.reshape(N).astype(jnp.int32),
      wt_pad,
      pos1_table.astype(jnp.float32).reshape(PLEN * PDIM),
      pos2_table.astype(jnp.float32).reshape(PLEN * PDIM),
  )
  return out.reshape(B, L, ODIM)


# speedup vs baseline: 1.2857x; 1.2857x over previous
import jax
import jax.numpy as jnp
from jax import lax
from jax.experimental import pallas as pl
from jax.experimental.pallas import tpu as pltpu
from jax.experimental.pallas import tpu_sc as plsc

B = 4096
L = 200
WDIM = 50
PDIM = 5
ODIM = 60
TPAD = 56
PLEN = 400

NC = 2
NS = 16
NW = NC * NS
ROWS_W = B // NW
R = 2
C = R * L
CHUNKS = ROWS_W // R


def _body(word_hbm, p1_hbm, p2_hbm, wt_hbm, p1t_hbm, p2t_hbm, out_hbm,
          widx, p1idx, p2idx, t56, t60, p1t, p2t, semi, semg, semo):
  wid = lax.axis_index("s") * NC + lax.axis_index("c")
  base = wid * ROWS_W
  pltpu.sync_copy(p1t_hbm, p1t)
  pltpu.sync_copy(p2t_hbm, p2t)

  def fire_idx(g, b):
    row = base + g * R
    pltpu.async_copy(word_hbm.at[pl.ds(row, R)], widx.at[b], semi[b])
    pltpu.async_copy(p1_hbm.at[pl.ds(row, R)], p1idx.at[b], semi[b])
    pltpu.async_copy(p2_hbm.at[pl.ds(row, R)], p2idx.at[b], semi[b])

  def wait_idx(b):
    pltpu.make_async_copy(word_hbm.at[pl.ds(0, R)], widx.at[b], semi[b]).wait()
    pltpu.make_async_copy(p1_hbm.at[pl.ds(0, R)], p1idx.at[b], semi[b]).wait()
    pltpu.make_async_copy(p2_hbm.at[pl.ds(0, R)], p2idx.at[b], semi[b]).wait()

  def fire_gathers(b):
    for r in range(R):
      pltpu.async_copy(wt_hbm.at[widx.at[b, r, pl.ds(0, 128)]],
                       t56.at[b, pl.ds(r * L, 128)], semg[b])
      pltpu.async_copy(wt_hbm.at[widx.at[b, r, pl.ds(128, L - 128)]],
                       t56.at[b, pl.ds(r * L + 128, L - 128)], semg[b])

  def wait_gathers(b):
    pltpu.make_async_copy(wt_hbm.at[pl.ds(0, C)], t56.at[b], semg[b]).wait()

  def fire_out(g, b):
    cb = pl.multiple_of((base + g * R) * L, 8)
    pltpu.async_copy(t60.at[b], out_hbm.at[pl.ds(cb, C)], semo[b])

  def wait_out(b):
    pltpu.make_async_copy(t60.at[b], out_hbm.at[pl.ds(0, C)], semo[b]).wait()

  fire_idx(0, 0)
  fire_idx(1, 1)
  wait_idx(0)
  fire_gathers(0)

  @pl.loop(0, CHUNKS // 2)
  def _outer(go):
    for b in range(2):
      g = go * 2 + b
      nb = 1 - b

      @pl.when(g + 1 < CHUNKS)
      def _():
        wait_idx(nb)
        fire_gathers(nb)

      wait_gathers(b)

      @pl.when(g >= 2)
      def _():
        wait_out(b)

      @plsc.parallel_loop(0, C, unroll=4)
      def _tok(t):
        v0 = t56.at[b, t][pl.ds(0, 16)]
        v1 = t56.at[b, t][pl.ds(16, 16)]
        v2 = t56.at[b, t][pl.ds(32, 16)]
        v3 = t56.at[b, t][pl.ds(40, 16)]
        t60.at[b, t][pl.ds(0, 16)] = v0
        t60.at[b, t][pl.ds(16, 16)] = v1
        t60.at[b, t][pl.ds(32, 16)] = v2
        tail = lax.iota(jnp.int32, 16) + 40
        plsc.store_scatter(t60.at[b], [jnp.full((16,), t, jnp.int32), tail], v3,
                           mask=(tail >= 48) & (tail < WDIM))

      for r in range(R):
        @plsc.parallel_loop(0, 13, unroll=2)
        def _grp(i):
          off = jnp.minimum(i * 16, L - 16)
          rows = lax.iota(jnp.int32, 16) + (r * L + off)
          i1 = p1idx.at[b, r][pl.ds(off, 16)] * PDIM
          i2 = p2idx.at[b, r][pl.ds(off, 16)] * PDIM
          v1s = [plsc.load_gather(p1t, [i1 + jnp.full((16,), j, jnp.int32)])
                 for j in range(PDIM)]
          v2s = [plsc.load_gather(p2t, [i2 + jnp.full((16,), j, jnp.int32)])
                 for j in range(PDIM)]
          for j in range(PDIM):
            plsc.store_scatter(
                t60.at[b], [rows, jnp.full((16,), WDIM + j, jnp.int32)], v1s[j])
            plsc.store_scatter(
                t60.at[b], [rows, jnp.full((16,), WDIM + PDIM + j, jnp.int32)], v2s[j])

      fire_out(g, b)

      @pl.when(g + 2 < CHUNKS)
      def _():
        fire_idx(g + 2, b)

  wait_out(0)
  wait_out(1)


def kernel(word, pos1, pos2, word_table, pos1_table, pos2_table):
  mesh = plsc.VectorSubcoreMesh(core_axis_name="c", subcore_axis_name="s")
  run = pl.kernel(
      _body,
      out_type=jax.ShapeDtypeStruct((B * L, ODIM), jnp.float32),
      mesh=mesh,
      scratch_types=[
          pltpu.VMEM((2, R, L), jnp.int32),
          pltpu.VMEM((2, R, L), jnp.int32),
          pltpu.VMEM((2, R, L), jnp.int32),
          pltpu.VMEM((2, C, TPAD), jnp.float32),
          pltpu.VMEM((2, C, ODIM), jnp.float32),
          pltpu.VMEM((PLEN * PDIM,), jnp.float32),
          pltpu.VMEM((PLEN * PDIM,), jnp.float32),
          [pltpu.SemaphoreType.DMA, pltpu.SemaphoreType.DMA],
          [pltpu.SemaphoreType.DMA, pltpu.SemaphoreType.DMA],
          [pltpu.SemaphoreType.DMA, pltpu.SemaphoreType.DMA],
      ],
      compiler_params=pltpu.CompilerParams(
          use_tc_tiling_on_sc=False, needs_layout_passes=False),
  )
  wt_pad = jnp.pad(word_table.astype(jnp.float32), ((0, 0), (0, TPAD - WDIM)))
  out = run(
      word.astype(jnp.int32),
      pos1.astype(jnp.int32),
      pos2.astype(jnp.int32),
      wt_pad,
      pos1_table.astype(jnp.float32).reshape(PLEN * PDIM),
      pos2_table.astype(jnp.float32).reshape(PLEN * PDIM),
  )
  return out.reshape(B, L, ODIM)


# speedup vs baseline: 1.3082x; 1.0175x over previous
import jax
import jax.numpy as jnp
from jax import lax
from jax.experimental import pallas as pl
from jax.experimental.pallas import tpu as pltpu
from jax.experimental.pallas import tpu_sc as plsc

B = 4096
L = 200
WDIM = 50
PDIM = 5
ODIM = 60
TPAD = 64
PLEN = 400

NC = 2
NS = 16
NW = NC * NS
ROWS_W = B // NW
R = 2
C = R * L
CHUNKS = ROWS_W // R


def _body(word_hbm, p1_hbm, p2_hbm, wt_hbm, p1t_hbm, p2t_hbm, out_hbm,
          widx, p1idx, p2idx, t56, t60, p1t, p2t, semi, semg, semo):
  wid = lax.axis_index("s") * NC + lax.axis_index("c")
  base = wid * ROWS_W
  pltpu.sync_copy(p1t_hbm, p1t)
  pltpu.sync_copy(p2t_hbm, p2t)

  def fire_idx(g, b):
    row = base + g * R
    pltpu.async_copy(word_hbm.at[pl.ds(row, R)], widx.at[b], semi[b])
    pltpu.async_copy(p1_hbm.at[pl.ds(row, R)], p1idx.at[b], semi[b])
    pltpu.async_copy(p2_hbm.at[pl.ds(row, R)], p2idx.at[b], semi[b])

  def wait_idx(b):
    pltpu.make_async_copy(word_hbm.at[pl.ds(0, R)], widx.at[b], semi[b]).wait()
    pltpu.make_async_copy(p1_hbm.at[pl.ds(0, R)], p1idx.at[b], semi[b]).wait()
    pltpu.make_async_copy(p2_hbm.at[pl.ds(0, R)], p2idx.at[b], semi[b]).wait()

  def fire_gathers(b):
    for r in range(R):
      pltpu.async_copy(wt_hbm.at[widx.at[b, r, pl.ds(0, 128)]],
                       t56.at[b, pl.ds(r * L, 128)], semg[b])
      pltpu.async_copy(wt_hbm.at[widx.at[b, r, pl.ds(128, L - 128)]],
                       t56.at[b, pl.ds(r * L + 128, L - 128)], semg[b])

  def wait_gathers(b):
    pltpu.make_async_copy(wt_hbm.at[pl.ds(0, C)], t56.at[b], semg[b]).wait()

  def fire_out(g, b):
    cb = pl.multiple_of((base + g * R) * L, 8)
    pltpu.async_copy(t60.at[b], out_hbm.at[pl.ds(cb, C)], semo[b])

  def wait_out(b):
    pltpu.make_async_copy(t60.at[b], out_hbm.at[pl.ds(0, C)], semo[b]).wait()

  fire_idx(0, 0)
  fire_idx(1, 1)
  wait_idx(0)
  fire_gathers(0)

  @pl.loop(0, CHUNKS // 2)
  def _outer(go):
    for b in range(2):
      g = go * 2 + b
      nb = 1 - b

      @pl.when(g + 1 < CHUNKS)
      def _():
        wait_idx(nb)
        fire_gathers(nb)

      wait_gathers(b)

      @pl.when(g >= 2)
      def _():
        wait_out(b)

      @plsc.parallel_loop(0, C, unroll=4)
      def _tok(t):
        v0 = t56.at[b, t][pl.ds(0, 16)]
        v1 = t56.at[b, t][pl.ds(16, 16)]
        v2 = t56.at[b, t][pl.ds(32, 16)]
        v3 = t56.at[b, t][pl.ds(40, 16)]
        t60.at[b, t][pl.ds(0, 16)] = v0
        t60.at[b, t][pl.ds(16, 16)] = v1
        t60.at[b, t][pl.ds(32, 16)] = v2
        tail = lax.iota(jnp.int32, 16) + 40
        plsc.store_scatter(t60.at[b], [jnp.full((16,), t, jnp.int32), tail], v3,
                           mask=(tail >= 48) & (tail < WDIM))

      for r in range(R):
        @plsc.parallel_loop(0, 13, unroll=2)
        def _grp(i):
          off = jnp.minimum(i * 16, L - 16)
          rows = lax.iota(jnp.int32, 16) + (r * L + off)
          i1 = p1idx.at[b, r][pl.ds(off, 16)] * PDIM
          i2 = p2idx.at[b, r][pl.ds(off, 16)] * PDIM
          v1s = [plsc.load_gather(p1t, [i1 + jnp.full((16,), j, jnp.int32)])
                 for j in range(PDIM)]
          v2s = [plsc.load_gather(p2t, [i2 + jnp.full((16,), j, jnp.int32)])
                 for j in range(PDIM)]
          for j in range(PDIM):
            plsc.store_scatter(
                t60.at[b], [rows, jnp.full((16,), WDIM + j, jnp.int32)], v1s[j])
            plsc.store_scatter(
                t60.at[b], [rows, jnp.full((16,), WDIM + PDIM + j, jnp.int32)], v2s[j])

      fire_out(g, b)

      @pl.when(g + 2 < CHUNKS)
      def _():
        fire_idx(g + 2, b)

  wait_out(0)
  wait_out(1)


def kernel(word, pos1, pos2, word_table, pos1_table, pos2_table):
  mesh = plsc.VectorSubcoreMesh(core_axis_name="c", subcore_axis_name="s")
  run = pl.kernel(
      _body,
      out_type=jax.ShapeDtypeStruct((B * L, ODIM), jnp.float32),
      mesh=mesh,
      scratch_types=[
          pltpu.VMEM((2, R, L), jnp.int32),
          pltpu.VMEM((2, R, L), jnp.int32),
          pltpu.VMEM((2, R, L), jnp.int32),
          pltpu.VMEM((2, C, TPAD), jnp.float32),
          pltpu.VMEM((2, C, ODIM), jnp.float32),
          pltpu.VMEM((PLEN * PDIM,), jnp.float32),
          pltpu.VMEM((PLEN * PDIM,), jnp.float32),
          [pltpu.SemaphoreType.DMA, pltpu.SemaphoreType.DMA],
          [pltpu.SemaphoreType.DMA, pltpu.SemaphoreType.DMA],
          [pltpu.SemaphoreType.DMA, pltpu.SemaphoreType.DMA],
      ],
      compiler_params=pltpu.CompilerParams(
          use_tc_tiling_on_sc=False, needs_layout_passes=False),
  )
  wt_pad = jnp.pad(word_table.astype(jnp.float32), ((0, 0), (0, TPAD - WDIM)))
  out = run(
      word.astype(jnp.int32),
      pos1.astype(jnp.int32),
      pos2.astype(jnp.int32),
      wt_pad,
      pos1_table.astype(jnp.float32).reshape(PLEN * PDIM),
      pos2_table.astype(jnp.float32).reshape(PLEN * PDIM),
  )
  return out.reshape(B, L, ODIM)


# speedup vs baseline: 1.3928x; 1.0647x over previous
import jax
import jax.numpy as jnp
from jax import lax
from jax.experimental import pallas as pl
from jax.experimental.pallas import tpu as pltpu
from jax.experimental.pallas import tpu_sc as plsc

B = 4096
L = 200
WDIM = 50
PDIM = 5
ODIM = 60
TPAD = 64
PLEN = 400

NC = 2
NS = 16
NW = NC * NS
ROWS_W = B // NW
R = 2
C = R * L
CHUNKS = ROWS_W // R


def _body(word_hbm, p1_hbm, p2_hbm, wt_hbm, p1t_hbm, p2t_hbm, out_hbm,
          widx, p1idx, p2idx, t64, p1t, p2t, semi, semg, semo):
  wid = lax.axis_index("s") * NC + lax.axis_index("c")
  base = wid * ROWS_W
  pltpu.sync_copy(p1t_hbm, p1t)
  pltpu.sync_copy(p2t_hbm, p2t)

  def fire_idx(g, b):
    row = base + g * R
    pltpu.async_copy(word_hbm.at[pl.ds(row, R)], widx.at[b], semi[b])
    pltpu.async_copy(p1_hbm.at[pl.ds(row, R)], p1idx.at[b], semi[b])
    pltpu.async_copy(p2_hbm.at[pl.ds(row, R)], p2idx.at[b], semi[b])

  def wait_idx(b):
    pltpu.make_async_copy(word_hbm.at[pl.ds(0, R)], widx.at[b], semi[b]).wait()
    pltpu.make_async_copy(p1_hbm.at[pl.ds(0, R)], p1idx.at[b], semi[b]).wait()
    pltpu.make_async_copy(p2_hbm.at[pl.ds(0, R)], p2idx.at[b], semi[b]).wait()

  def fire_gathers(b):
    for r in range(R):
      pltpu.async_copy(wt_hbm.at[widx.at[b, r, pl.ds(0, 128)]],
                       t64.at[b, pl.ds(r * L, 128)], semg[b])
      pltpu.async_copy(wt_hbm.at[widx.at[b, r, pl.ds(128, L - 128)]],
                       t64.at[b, pl.ds(r * L + 128, L - 128)], semg[b])

  def wait_gathers(b):
    pltpu.make_async_copy(wt_hbm.at[pl.ds(0, C)], t64.at[b], semg[b]).wait()

  def fire_out(g, b):
    cb = pl.multiple_of((base + g * R) * L, 8)
    pltpu.async_copy(t64.at[b], out_hbm.at[pl.ds(cb, C)], semo[b])

  def wait_out(b):
    pltpu.make_async_copy(t64.at[b], out_hbm.at[pl.ds(0, C)], semo[b]).wait()

  fire_idx(0, 0)
  fire_idx(1, 1)
  wait_idx(0)
  fire_gathers(0)

  @pl.loop(0, CHUNKS // 2)
  def _outer(go):
    for b in range(2):
      g = go * 2 + b
      nb = 1 - b

      @pl.when((g >= 1) & (g + 1 < CHUNKS))
      def _():
        wait_out(nb)

      @pl.when(g + 1 < CHUNKS)
      def _():
        wait_idx(nb)
        fire_gathers(nb)

      wait_gathers(b)

      for r in range(R):
        @plsc.parallel_loop(0, 13, unroll=2)
        def _grp(i):
          off = jnp.minimum(i * 16, L - 16)
          rows = lax.iota(jnp.int32, 16) + (r * L + off)
          i1 = p1idx.at[b, r][pl.ds(off, 16)] * PDIM
          i2 = p2idx.at[b, r][pl.ds(off, 16)] * PDIM
          v1s = [plsc.load_gather(p1t, [i1 + jnp.full((16,), j, jnp.int32)])
                 for j in range(PDIM)]
          v2s = [plsc.load_gather(p2t, [i2 + jnp.full((16,), j, jnp.int32)])
                 for j in range(PDIM)]
          for j in range(PDIM):
            plsc.store_scatter(
                t64.at[b], [rows, jnp.full((16,), WDIM + j, jnp.int32)], v1s[j])
            plsc.store_scatter(
                t64.at[b], [rows, jnp.full((16,), WDIM + PDIM + j, jnp.int32)], v2s[j])

      fire_out(g, b)

      @pl.when(g + 2 < CHUNKS)
      def _():
        fire_idx(g + 2, b)

  wait_out(1)


def kernel(word, pos1, pos2, word_table, pos1_table, pos2_table):
  mesh = plsc.VectorSubcoreMesh(core_axis_name="c", subcore_axis_name="s")
  run = pl.kernel(
      _body,
      out_type=jax.ShapeDtypeStruct((B * L, TPAD), jnp.float32),
      mesh=mesh,
      scratch_types=[
          pltpu.VMEM((2, R, L), jnp.int32),
          pltpu.VMEM((2, R, L), jnp.int32),
          pltpu.VMEM((2, R, L), jnp.int32),
          pltpu.VMEM((2, C, TPAD), jnp.float32),
          pltpu.VMEM((PLEN * PDIM,), jnp.float32),
          pltpu.VMEM((PLEN * PDIM,), jnp.float32),
          [pltpu.SemaphoreType.DMA, pltpu.SemaphoreType.DMA],
          [pltpu.SemaphoreType.DMA, pltpu.SemaphoreType.DMA],
          [pltpu.SemaphoreType.DMA, pltpu.SemaphoreType.DMA],
      ],
      compiler_params=pltpu.CompilerParams(
          use_tc_tiling_on_sc=False, needs_layout_passes=False),
  )
  wt_pad = jnp.pad(word_table.astype(jnp.float32), ((0, 0), (0, TPAD - WDIM)))
  out = run(
      word.astype(jnp.int32),
      pos1.astype(jnp.int32),
      pos2.astype(jnp.int32),
      wt_pad,
      pos1_table.astype(jnp.float32).reshape(PLEN * PDIM),
      pos2_table.astype(jnp.float32).reshape(PLEN * PDIM),
  )
  return out[:, :ODIM].reshape(B, L, ODIM)


# speedup vs baseline: 1.4085x; 1.0113x over previous
import jax
import jax.numpy as jnp
from jax import lax
from jax.experimental import pallas as pl
from jax.experimental.pallas import tpu as pltpu
from jax.experimental.pallas import tpu_sc as plsc

B = 4096
L = 200
WDIM = 50
PDIM = 5
ODIM = 60
TPAD = 64
PLEN = 400

NC = 2
NS = 16
NW = NC * NS
ROWS_W = B // NW
R = 4
C = R * L
CHUNKS = ROWS_W // R


def _body(word_hbm, p1_hbm, p2_hbm, wt_hbm, p1t_hbm, p2t_hbm, out_hbm,
          widx, p1idx, p2idx, t64, p1t, p2t, semi, semg, semo):
  wid = lax.axis_index("s") * NC + lax.axis_index("c")
  base = wid * ROWS_W
  pltpu.sync_copy(p1t_hbm, p1t)
  pltpu.sync_copy(p2t_hbm, p2t)

  def fire_idx(g, b):
    row = base + g * R
    pltpu.async_copy(word_hbm.at[pl.ds(row, R)], widx.at[b], semi[b])
    pltpu.async_copy(p1_hbm.at[pl.ds(row, R)], p1idx.at[b], semi[b])
    pltpu.async_copy(p2_hbm.at[pl.ds(row, R)], p2idx.at[b], semi[b])

  def wait_idx(b):
    pltpu.make_async_copy(word_hbm.at[pl.ds(0, R)], widx.at[b], semi[b]).wait()
    pltpu.make_async_copy(p1_hbm.at[pl.ds(0, R)], p1idx.at[b], semi[b]).wait()
    pltpu.make_async_copy(p2_hbm.at[pl.ds(0, R)], p2idx.at[b], semi[b]).wait()

  def fire_gathers(b):
    for r in range(R):
      pltpu.async_copy(wt_hbm.at[widx.at[b, r, pl.ds(0, 128)]],
                       t64.at[b, pl.ds(r * L, 128)], semg[b])
      pltpu.async_copy(wt_hbm.at[widx.at[b, r, pl.ds(128, L - 128)]],
                       t64.at[b, pl.ds(r * L + 128, L - 128)], semg[b])

  def wait_gathers(b):
    pltpu.make_async_copy(wt_hbm.at[pl.ds(0, C)], t64.at[b], semg[b]).wait()

  def fire_out(g, b):
    cb = pl.multiple_of((base + g * R) * L, 8)
    pltpu.async_copy(t64.at[b], out_hbm.at[pl.ds(cb, C)], semo[b])

  def wait_out(b):
    pltpu.make_async_copy(t64.at[b], out_hbm.at[pl.ds(0, C)], semo[b]).wait()

  fire_idx(0, 0)
  fire_idx(1, 1)
  wait_idx(0)
  fire_gathers(0)

  @pl.loop(0, CHUNKS // 2)
  def _outer(go):
    for b in range(2):
      g = go * 2 + b
      nb = 1 - b

      @pl.when((g >= 1) & (g + 1 < CHUNKS))
      def _():
        wait_out(nb)

      @pl.when(g + 1 < CHUNKS)
      def _():
        wait_idx(nb)
        fire_gathers(nb)

      wait_gathers(b)

      for r in range(R):
        @plsc.parallel_loop(0, 13, unroll=2)
        def _grp(i):
          off = jnp.minimum(i * 16, L - 16)
          rows = lax.iota(jnp.int32, 16) + (r * L + off)
          i1 = p1idx.at[b, r][pl.ds(off, 16)] * PDIM
          i2 = p2idx.at[b, r][pl.ds(off, 16)] * PDIM
          v1s = [plsc.load_gather(p1t, [i1 + jnp.full((16,), j, jnp.int32)])
                 for j in range(PDIM)]
          v2s = [plsc.load_gather(p2t, [i2 + jnp.full((16,), j, jnp.int32)])
                 for j in range(PDIM)]
          for j in range(PDIM):
            plsc.store_scatter(
                t64.at[b], [rows, jnp.full((16,), WDIM + j, jnp.int32)], v1s[j])
            plsc.store_scatter(
                t64.at[b], [rows, jnp.full((16,), WDIM + PDIM + j, jnp.int32)], v2s[j])

      fire_out(g, b)

      @pl.when(g + 2 < CHUNKS)
      def _():
        fire_idx(g + 2, b)

  wait_out(1)


def kernel(word, pos1, pos2, word_table, pos1_table, pos2_table):
  mesh = plsc.VectorSubcoreMesh(core_axis_name="c", subcore_axis_name="s")
  run = pl.kernel(
      _body,
      out_type=jax.ShapeDtypeStruct((B * L, TPAD), jnp.float32),
      mesh=mesh,
      scratch_types=[
          pltpu.VMEM((2, R, L), jnp.int32),
          pltpu.VMEM((2, R, L), jnp.int32),
          pltpu.VMEM((2, R, L), jnp.int32),
          pltpu.VMEM((2, C, TPAD), jnp.float32),
          pltpu.VMEM((PLEN * PDIM,), jnp.float32),
          pltpu.VMEM((PLEN * PDIM,), jnp.float32),
          [pltpu.SemaphoreType.DMA, pltpu.SemaphoreType.DMA],
          [pltpu.SemaphoreType.DMA, pltpu.SemaphoreType.DMA],
          [pltpu.SemaphoreType.DMA, pltpu.SemaphoreType.DMA],
      ],
      compiler_params=pltpu.CompilerParams(
          use_tc_tiling_on_sc=False, needs_layout_passes=False),
  )
  wt_pad = jnp.pad(word_table.astype(jnp.float32), ((0, 0), (0, TPAD - WDIM)))
  out = run(
      word.astype(jnp.int32),
      pos1.astype(jnp.int32),
      pos2.astype(jnp.int32),
      wt_pad,
      pos1_table.astype(jnp.float32).reshape(PLEN * PDIM),
      pos2_table.astype(jnp.float32).reshape(PLEN * PDIM),
  )
  return out[:, :ODIM].reshape(B, L, ODIM)
